# baseline (speedup 1.0000x reference)
import functools

import jax
import jax.numpy as jnp
from jax import lax
from jax.experimental import pallas as pl
from jax.experimental.pallas import tpu as pltpu
from jax.experimental.pallas import tpu_sc as plsc

BATCH = 16384
EMBED = 64
NUM_DATASETS = 1000000

NUM_CORES = 2
NUM_SUBCORES = 16
NUM_WORKERS = NUM_CORES * NUM_SUBCORES

SCCOLS = 262144
TCCOLS = NUM_DATASETS - SCCOLS
BC = 32768
TCOFF = SCCOLS // BC
GRID = (TCCOLS + BC - 1) // BC
SCHUNK = 512
CPT = SCCOLS // NUM_WORKERS
NSCHUNK = CPT // SCHUNK

B_PER_W = BATCH // NUM_WORKERS
CHUNK = 128
NCHUNK = B_PER_W // CHUNK

_mesh = plsc.VectorSubcoreMesh(core_axis_name="c", subcore_axis_name="s")


def _mv_body(w_ref, b_ref, t_ref, o_ref):
    x = t_ref[...]
    w = w_ref[...].reshape(EMBED, 1)
    o_ref[...] = (x * w).sum(axis=0) + b_ref[0]


_tc_matvec = pl.pallas_call(
    _mv_body,
    grid=(GRID,),
    in_specs=[
        pl.BlockSpec((1, EMBED), lambda i: (0, 0)),
        pl.BlockSpec(memory_space=pltpu.SMEM),
        pl.BlockSpec((EMBED, BC), lambda i: (0, i + TCOFF)),
    ],
    out_specs=pl.BlockSpec((BC,), lambda i: (i,)),
    out_shape=jax.ShapeDtypeStruct((TCCOLS,), jnp.float32),
)


@functools.partial(
    pl.kernel,
    out_type=jax.ShapeDtypeStruct((SCCOLS,), jnp.float32),
    mesh=_mesh,
    compiler_params=pltpu.CompilerParams(needs_layout_passes=False),
    scratch_types=[
        [pltpu.VMEM((EMBED, SCHUNK), jnp.float32) for _ in range(2)],
        pltpu.VMEM((SCHUNK,), jnp.float32),
        pltpu.VMEM((EMBED,), jnp.float32),
        pltpu.VMEM((16,), jnp.float32),
        pltpu.SemaphoreType.DMA,
    ],
)
def _sc_matvec(tablet_hbm, w_hbm, b_hbm, ssc_hbm, xbuf, y_v, w_v, b_v, sem):
    wid = lax.axis_index("s") * NUM_CORES + lax.axis_index("c")

    pltpu.sync_copy(w_hbm.at[0], w_v)
    pltpu.sync_copy(b_hbm, b_v.at[pl.ds(0, 1)])
    b_s = b_v[pl.ds(0, 16)][0]
    w_vecs = [w_v[pl.ds(k * 16, 16)] for k in range(EMBED // 16)]
    w_s = [w_vecs[d // 16][d % 16] for d in range(EMBED)]

    def fetch(k):
        off = pl.multiple_of(wid * CPT + k * SCHUNK, SCHUNK)
        return pltpu.async_copy(
            tablet_hbm.at[:, pl.ds(off, SCHUNK)], xbuf[k % 2], sem)

    pending = fetch(0)
    for k in range(NSCHUNK):
        pending.wait()
        if k + 1 < NSCHUNK:
            pending = fetch(k + 1)
        xb = xbuf[k % 2]

        def body(g, _, xb=xb):
            goff = pl.multiple_of(g * 16, 16)
            acc = jnp.full((16,), b_s, jnp.float32)
            for d in range(EMBED):
                acc = acc + xb[d, pl.ds(goff, 16)] * w_s[d]
            y_v[pl.ds(goff, 16)] = acc
            return 0

        lax.fori_loop(0, SCHUNK // 16, body, 0)
        pltpu.sync_copy(y_v, ssc_hbm.at[pl.ds(wid * CPT + k * SCHUNK, SCHUNK)])


@functools.partial(
    pl.kernel,
    out_type=jax.ShapeDtypeStruct((BATCH,), jnp.float32),
    mesh=_mesh,
    compiler_params=pltpu.CompilerParams(
        needs_layout_passes=False, use_tc_tiling_on_sc=False),
    scratch_types=[
        pltpu.VMEM((NCHUNK, CHUNK), jnp.int32),
        pltpu.VMEM((NCHUNK, CHUNK), jnp.int32),
        pltpu.VMEM((NCHUNK, CHUNK), jnp.int32),
        pltpu.VMEM((B_PER_W,), jnp.float32),
        pltpu.VMEM((B_PER_W,), jnp.float32),
        pltpu.VMEM((B_PER_W,), jnp.float32),
        pltpu.SemaphoreType.DMA,
    ],
)
def _sc_gather(ids_hbm, stc_hbm, ssc_hbm, out_hbm,
               idx_v, idxa_v, idxb_v, va_v, vb_v, out_v, sem):
    wid = lax.axis_index("s") * NUM_CORES + lax.axis_index("c")
    base = pl.multiple_of(wid * B_PER_W, B_PER_W)

    for c in range(NCHUNK):
        pltpu.sync_copy(ids_hbm.at[pl.ds(base + c * CHUNK, CHUNK)],
                        idx_v.at[c])
    copies = []
    for c in range(NCHUNK):
        def cbody(j, _, c=c):
            joff = pl.multiple_of(j * 16, 16)
            ids16 = idx_v[c, pl.ds(joff, 16)]
            idxa_v[c, pl.ds(joff, 16)] = jnp.maximum(ids16 - SCCOLS, 0)
            idxb_v[c, pl.ds(joff, 16)] = jnp.minimum(ids16, SCCOLS - 1)
            return 0
        lax.fori_loop(0, CHUNK // 16, cbody, 0)
        copies.append(
            pltpu.async_copy(stc_hbm.at[idxa_v.at[c]],
                             va_v.at[pl.ds(c * CHUNK, CHUNK)], sem))
        copies.append(
            pltpu.async_copy(ssc_hbm.at[idxb_v.at[c]],
                             vb_v.at[pl.ds(c * CHUNK, CHUNK)], sem))
    for cp in copies:
        cp.wait()

    for c in range(NCHUNK):
        def mbody(j, _, c=c):
            joff = pl.multiple_of(j * 16, 16)
            ids16 = idx_v[c, pl.ds(joff, 16)]
            va = va_v[pl.ds(c * CHUNK + joff, 16)]
            vb = vb_v[pl.ds(c * CHUNK + joff, 16)]
            out_v[pl.ds(c * CHUNK + joff, 16)] = jnp.where(
                ids16 >= SCCOLS, va, vb)
            return 0
        lax.fori_loop(0, CHUNK // 16, mbody, 0)

    pltpu.sync_copy(out_v, out_hbm.at[pl.ds(base, B_PER_W)])


def kernel(dataset_ids, table, W, b):
    tablet = table.T
    s_sc = _sc_matvec(tablet, W, b)
    s_tc = _tc_matvec(W, b, tablet)
    return _sc_gather(dataset_ids.astype(jnp.int32), s_tc, s_sc)


# speedup vs baseline: 1.4993x; 1.4993x over previous
import functools

import jax
import jax.numpy as jnp
from jax import lax
from jax.experimental import pallas as pl
from jax.experimental.pallas import tpu as pltpu
from jax.experimental.pallas import tpu_sc as plsc

BATCH = 16384
EMBED = 64
NUM_DATASETS = 1000000

NUM_CORES = 2
NUM_SUBCORES = 16
NUM_WORKERS = NUM_CORES * NUM_SUBCORES

SCCOLS = 262144
TCCOLS = NUM_DATASETS - SCCOLS
BC = 32768
TCOFF = SCCOLS // BC
GRID = (TCCOLS + BC - 1) // BC
SCHUNK = 512
CPT = SCCOLS // NUM_WORKERS
NSCHUNK = CPT // SCHUNK

B_PER_W = BATCH // NUM_WORKERS
CHUNK = 128
NCHUNK = B_PER_W // CHUNK

_mesh = plsc.VectorSubcoreMesh(core_axis_name="c", subcore_axis_name="s")


def _mv_body(w_ref, b_ref, t_ref, o_ref):
    x = t_ref[...]
    w = w_ref[...].reshape(EMBED, 1)
    o_ref[...] = (x * w).sum(axis=0) + b_ref[0]


_tc_matvec = pl.pallas_call(
    _mv_body,
    grid=(GRID,),
    in_specs=[
        pl.BlockSpec((1, EMBED), lambda i: (0, 0)),
        pl.BlockSpec(memory_space=pltpu.SMEM),
        pl.BlockSpec((EMBED, BC), lambda i: (0, i + TCOFF)),
    ],
    out_specs=pl.BlockSpec((BC,), lambda i: (i,)),
    out_shape=jax.ShapeDtypeStruct((TCCOLS,), jnp.float32),
)


@functools.partial(
    pl.kernel,
    out_type=jax.ShapeDtypeStruct((SCCOLS,), jnp.float32),
    mesh=_mesh,
    compiler_params=pltpu.CompilerParams(needs_layout_passes=False),
    scratch_types=[
        [pltpu.VMEM((EMBED, SCHUNK), jnp.float32) for _ in range(2)],
        pltpu.VMEM((SCHUNK,), jnp.float32),
        pltpu.VMEM((EMBED,), jnp.float32),
        pltpu.VMEM((16,), jnp.float32),
        pltpu.SemaphoreType.DMA,
    ],
)
def _sc_matvec(tablet_hbm, w_hbm, b_hbm, ssc_hbm, xbuf, y_v, w_v, b_v, sem):
    wid = lax.axis_index("s") * NUM_CORES + lax.axis_index("c")

    pltpu.sync_copy(w_hbm.at[0], w_v)
    pltpu.sync_copy(b_hbm, b_v.at[pl.ds(0, 1)])
    b_s = b_v[pl.ds(0, 16)][0]
    w_vecs = [w_v[pl.ds(k * 16, 16)] for k in range(EMBED // 16)]
    w_s = [w_vecs[d // 16][d % 16] for d in range(EMBED)]

    def fetch(k):
        off = pl.multiple_of(wid * CPT + k * SCHUNK, SCHUNK)
        return pltpu.async_copy(
            tablet_hbm.at[:, pl.ds(off, SCHUNK)], xbuf[k % 2], sem)

    pending = fetch(0)
    for k in range(NSCHUNK):
        pending.wait()
        if k + 1 < NSCHUNK:
            pending = fetch(k + 1)
        xb = xbuf[k % 2]

        def body(g, _, xb=xb):
            goff = pl.multiple_of(g * 16, 16)
            accs = [jnp.full((16,), b_s, jnp.float32),
                    jnp.zeros((16,), jnp.float32),
                    jnp.zeros((16,), jnp.float32),
                    jnp.zeros((16,), jnp.float32)]
            for d in range(EMBED):
                accs[d % 4] = accs[d % 4] + xb[d, pl.ds(goff, 16)] * w_s[d]
            y_v[pl.ds(goff, 16)] = (accs[0] + accs[1]) + (accs[2] + accs[3])
            return 0

        lax.fori_loop(0, SCHUNK // 16, body, 0)
        pltpu.sync_copy(y_v, ssc_hbm.at[pl.ds(wid * CPT + k * SCHUNK, SCHUNK)])


@functools.partial(
    pl.kernel,
    out_type=jax.ShapeDtypeStruct((BATCH,), jnp.float32),
    mesh=_mesh,
    compiler_params=pltpu.CompilerParams(
        needs_layout_passes=False, use_tc_tiling_on_sc=False),
    scratch_types=[
        pltpu.VMEM((NCHUNK, CHUNK), jnp.int32),
        pltpu.VMEM((B_PER_W,), jnp.float32),
        pltpu.SemaphoreType.DMA,
    ],
)
def _sc_gather(ids_hbm, s_hbm, out_hbm, idx_v, out_v, sem):
    wid = lax.axis_index("s") * NUM_CORES + lax.axis_index("c")
    base = pl.multiple_of(wid * B_PER_W, B_PER_W)

    for c in range(NCHUNK):
        pltpu.sync_copy(ids_hbm.at[pl.ds(base + c * CHUNK, CHUNK)],
                        idx_v.at[c])
    copies = []
    for c in range(NCHUNK):
        copies.append(
            pltpu.async_copy(s_hbm.at[idx_v.at[c]],
                             out_v.at[pl.ds(c * CHUNK, CHUNK)], sem))
    for cp in copies:
        cp.wait()
    pltpu.sync_copy(out_v, out_hbm.at[pl.ds(base, B_PER_W)])


def kernel(dataset_ids, table, W, b):
    tablet = table.T
    s_sc = _sc_matvec(tablet, W, b)
    s_tc = _tc_matvec(W, b, tablet)
    s = jnp.concatenate([s_sc, s_tc])
    return _sc_gather(dataset_ids.astype(jnp.int32), s)


# speedup vs baseline: 1.5072x; 1.0052x over previous
import functools

import jax
import jax.numpy as jnp
from jax import lax
from jax.experimental import pallas as pl
from jax.experimental.pallas import tpu as pltpu
from jax.experimental.pallas import tpu_sc as plsc

BATCH = 16384
EMBED = 64
NUM_DATASETS = 1000000

NUM_CORES = 2
NUM_SUBCORES = 16
NUM_WORKERS = NUM_CORES * NUM_SUBCORES

SCCOLS = 131072
TCCOLS = NUM_DATASETS - SCCOLS
BC = 32768
TCOFF = SCCOLS // BC
GRID = (TCCOLS + BC - 1) // BC
SCHUNK = 512
CPT = SCCOLS // NUM_WORKERS
NSCHUNK = CPT // SCHUNK

B_PER_W = BATCH // NUM_WORKERS
CHUNK = 128
NCHUNK = B_PER_W // CHUNK

_mesh = plsc.VectorSubcoreMesh(core_axis_name="c", subcore_axis_name="s")


def _mv_body(w_ref, b_ref, t_ref, o_ref):
    x = t_ref[...]
    w = w_ref[...].reshape(EMBED, 1)
    o_ref[...] = (x * w).sum(axis=0) + b_ref[0]


_tc_matvec = pl.pallas_call(
    _mv_body,
    grid=(GRID,),
    in_specs=[
        pl.BlockSpec((1, EMBED), lambda i: (0, 0)),
        pl.BlockSpec(memory_space=pltpu.SMEM),
        pl.BlockSpec((EMBED, BC), lambda i: (0, i + TCOFF)),
    ],
    out_specs=pl.BlockSpec((BC,), lambda i: (i,)),
    out_shape=jax.ShapeDtypeStruct((TCCOLS,), jnp.float32),
)


@functools.partial(
    pl.kernel,
    out_type=jax.ShapeDtypeStruct((SCCOLS,), jnp.float32),
    mesh=_mesh,
    compiler_params=pltpu.CompilerParams(needs_layout_passes=False),
    scratch_types=[
        [pltpu.VMEM((EMBED, SCHUNK), jnp.float32) for _ in range(2)],
        pltpu.VMEM((SCHUNK,), jnp.float32),
        pltpu.VMEM((EMBED,), jnp.float32),
        pltpu.VMEM((16,), jnp.float32),
        pltpu.SemaphoreType.DMA,
    ],
)
def _sc_matvec(tablet_hbm, w_hbm, b_hbm, ssc_hbm, xbuf, y_v, w_v, b_v, sem):
    wid = lax.axis_index("s") * NUM_CORES + lax.axis_index("c")

    pltpu.sync_copy(w_hbm.at[0], w_v)
    pltpu.sync_copy(b_hbm, b_v.at[pl.ds(0, 1)])
    b_s = b_v[pl.ds(0, 16)][0]
    w_vecs = [w_v[pl.ds(k * 16, 16)] for k in range(EMBED // 16)]
    w_s = [w_vecs[d // 16][d % 16] for d in range(EMBED)]

    def fetch(k):
        off = pl.multiple_of(wid * CPT + k * SCHUNK, SCHUNK)
        return pltpu.async_copy(
            tablet_hbm.at[:, pl.ds(off, SCHUNK)], xbuf[k % 2], sem)

    pending = fetch(0)
    for k in range(NSCHUNK):
        pending.wait()
        if k + 1 < NSCHUNK:
            pending = fetch(k + 1)
        xb = xbuf[k % 2]

        def body(g, _, xb=xb):
            goff = pl.multiple_of(g * 16, 16)
            accs = [jnp.full((16,), b_s, jnp.float32),
                    jnp.zeros((16,), jnp.float32),
                    jnp.zeros((16,), jnp.float32),
                    jnp.zeros((16,), jnp.float32)]
            for d in range(EMBED):
                accs[d % 4] = accs[d % 4] + xb[d, pl.ds(goff, 16)] * w_s[d]
            y_v[pl.ds(goff, 16)] = (accs[0] + accs[1]) + (accs[2] + accs[3])
            return 0

        lax.fori_loop(0, SCHUNK // 16, body, 0)
        pltpu.sync_copy(y_v, ssc_hbm.at[pl.ds(wid * CPT + k * SCHUNK, SCHUNK)])


@functools.partial(
    pl.kernel,
    out_type=jax.ShapeDtypeStruct((BATCH,), jnp.float32),
    mesh=_mesh,
    compiler_params=pltpu.CompilerParams(
        needs_layout_passes=False, use_tc_tiling_on_sc=False),
    scratch_types=[
        pltpu.VMEM((NCHUNK, CHUNK), jnp.int32),
        pltpu.VMEM((B_PER_W,), jnp.float32),
        pltpu.SemaphoreType.DMA,
    ],
)
def _sc_gather(ids_hbm, s_hbm, out_hbm, idx_v, out_v, sem):
    wid = lax.axis_index("s") * NUM_CORES + lax.axis_index("c")
    base = pl.multiple_of(wid * B_PER_W, B_PER_W)

    for c in range(NCHUNK):
        pltpu.sync_copy(ids_hbm.at[pl.ds(base + c * CHUNK, CHUNK)],
                        idx_v.at[c])
    copies = []
    for c in range(NCHUNK):
        copies.append(
            pltpu.async_copy(s_hbm.at[idx_v.at[c]],
                             out_v.at[pl.ds(c * CHUNK, CHUNK)], sem))
    for cp in copies:
        cp.wait()
    pltpu.sync_copy(out_v, out_hbm.at[pl.ds(base, B_PER_W)])


def kernel(dataset_ids, table, W, b):
    tablet = table.T
    s_sc = _sc_matvec(tablet, W, b)
    s_tc = _tc_matvec(W, b, tablet)
    s = jnp.concatenate([s_sc, s_tc])
    return _sc_gather(dataset_ids.astype(jnp.int32), s)


# speedup vs baseline: 1.5624x; 1.0366x over previous
import functools

import jax
import jax.numpy as jnp
from jax import lax
from jax.experimental import pallas as pl
from jax.experimental.pallas import tpu as pltpu
from jax.experimental.pallas import tpu_sc as plsc

BATCH = 16384
EMBED = 64
NUM_DATASETS = 1000000
BC = 40960
GRID = (NUM_DATASETS + BC - 1) // BC

NUM_CORES = 2
NUM_SUBCORES = 16
NUM_WORKERS = NUM_CORES * NUM_SUBCORES
B_PER_W = BATCH // NUM_WORKERS
CHUNK = 128
NCHUNK = B_PER_W // CHUNK


def _mv_body(w_ref, b_ref, t_ref, o_ref):
    x = t_ref[...]
    w = w_ref[...].reshape(EMBED, 1)
    o_ref[...] = (x * w).sum(axis=0) + b_ref[0]


_matvec = pl.pallas_call(
    _mv_body,
    grid=(GRID,),
    in_specs=[
        pl.BlockSpec((1, EMBED), lambda i: (0, 0)),
        pl.BlockSpec(memory_space=pltpu.SMEM),
        pl.BlockSpec((EMBED, BC), lambda i: (0, i)),
    ],
    out_specs=pl.BlockSpec((BC,), lambda i: (i,)),
    out_shape=jax.ShapeDtypeStruct((NUM_DATASETS,), jnp.float32),
)

_mesh = plsc.VectorSubcoreMesh(core_axis_name="c", subcore_axis_name="s")


@functools.partial(
    pl.kernel,
    out_type=jax.ShapeDtypeStruct((BATCH,), jnp.float32),
    mesh=_mesh,
    compiler_params=pltpu.CompilerParams(
        needs_layout_passes=False, use_tc_tiling_on_sc=False),
    scratch_types=[
        pltpu.VMEM((NCHUNK, CHUNK), jnp.int32),
        pltpu.VMEM((B_PER_W,), jnp.float32),
        pltpu.SemaphoreType.DMA,
    ],
)
def _sc_gather(ids_hbm, s_hbm, out_hbm, idx_v, out_v, sem):
    wid = lax.axis_index("s") * NUM_CORES + lax.axis_index("c")
    base = pl.multiple_of(wid * B_PER_W, B_PER_W)

    for c in range(NCHUNK):
        pltpu.sync_copy(ids_hbm.at[pl.ds(base + c * CHUNK, CHUNK)],
                        idx_v.at[c])
    copies = []
    for c in range(NCHUNK):
        copies.append(
            pltpu.async_copy(s_hbm.at[idx_v.at[c]],
                             out_v.at[pl.ds(c * CHUNK, CHUNK)], sem))
    for cp in copies:
        cp.wait()
    pltpu.sync_copy(out_v, out_hbm.at[pl.ds(base, B_PER_W)])


def kernel(dataset_ids, table, W, b):
    s = _matvec(W, b, table.T)
    return _sc_gather(dataset_ids.astype(jnp.int32), s)
